# baseline (speedup 1.0000x reference)
import functools

import jax
import jax.numpy as jnp
from jax import lax
from jax.experimental import pallas as pl
from jax.experimental.pallas import tpu as pltpu
from jax.experimental.pallas import tpu_sc as plsc

N_NODES = 61440
N_GRAPHS = 1024
N_EDGES = 614400
FEAT = 128
MAXN = 120

NSUB = 16
LANE = 128
IDXW = 128
EROWS = N_EDGES // (NSUB * IDXW)
NSLICE = 3
SROWS = EROWS // NSLICE
NBUF = 5
DIDXW = 128
DROWS = N_EDGES // (2 * NSUB * DIDXW)
NSLAB = N_NODES // NSUB
CW = 16
NCH = FEAT // CW
ROW_TILE = 512
N_TILES = N_NODES // ROW_TILE
DEG_W = 16

_f32 = jnp.float32
_SC_PARAMS = pltpu.CompilerParams(use_tc_tiling_on_sc=False)


def _deg_body(dst_hbm, ones_hbm, zeros_hbm, deg_out, dst_v, ones_v, spm):
    c = lax.axis_index("c")
    s = lax.axis_index("s")
    w = c * NSUB + s
    pltpu.sync_copy(ones_hbm, ones_v)
    pltpu.sync_copy(dst_hbm.at[pl.ds(w * DROWS, DROWS)], dst_v)
    pltpu.sync_copy(zeros_hbm.at[pl.ds(s * NSLAB, NSLAB)],
                    spm.at[pl.ds(s * NSLAB, NSLAB)])
    plsc.subcore_barrier()

    @pl.loop(0, DROWS)
    def _(j):
        pltpu.sync_copy(ones_v, spm.at[dst_v.at[j]], add=True)

    plsc.subcore_barrier()
    pltpu.sync_copy(spm.at[pl.ds(s * NSLAB, NSLAB)],
                    deg_out.at[c, pl.ds(s * NSLAB, NSLAB)])


def _sc_deg(dst_lin, ones16, zeros16):
    fn = pl.kernel(
        _deg_body,
        out_type=jax.ShapeDtypeStruct((2, N_NODES, DEG_W), _f32),
        mesh=plsc.VectorSubcoreMesh(core_axis_name="c", subcore_axis_name="s"),
        scratch_types=[
            pltpu.VMEM((DROWS, DIDXW), jnp.int32),
            pltpu.VMEM((DIDXW, DEG_W), _f32),
            pltpu.VMEM_SHARED((N_NODES, DEG_W), _f32),
        ],
        compiler_params=_SC_PARAMS,
    )
    return fn(dst_lin, ones16, zeros16)


def _agg_body(hs, src_hbm, dst_hbm, agg, st0, st1, st2, st3, st4, st5, st6,
              st7,
              src_a, src_b, dst_a, dst_b, r0, r1, r2, r3, r4,
              sg0, sg1, sg2, sg3, sg4, ss0, ss1, ss2, ss3, ss4, si_a, si_b,
              spm):
    c = lax.axis_index("c")
    s = lax.axis_index("s")
    slab = pl.ds(s * NSLAB, NSLAB)
    stages = (st0, st1, st2, st3, st4, st5, st6, st7)
    srcs = (src_a, src_b)
    dsts = (dst_a, dst_b)
    isems = (si_a, si_b)
    rows = (r0, r1, r2, r3, r4)
    gsems = (sg0, sg1, sg2, sg3, sg4)
    ssems = (ss0, ss1, ss2, ss3, ss4)

    def idx_fetch(t, p):
        base = pl.ds(s * EROWS + t * SROWS, SROWS)
        pltpu.async_copy(src_hbm.at[base], srcs[p], isems[p])
        pltpu.async_copy(dst_hbm.at[base], dsts[p], isems[p])

    def idx_wait(p):
        pltpu.make_async_copy(src_hbm.at[pl.ds(0, SROWS)], srcs[p],
                              isems[p]).wait()
        pltpu.make_async_copy(dst_hbm.at[pl.ds(0, SROWS)], dsts[p],
                              isems[p]).wait()

    def run_chunk(q):
        col = pl.ds(q * CW, CW)
        hs_q = stages[q]
        pltpu.sync_copy(hs.at[slab, col], spm.at[slab])
        pltpu.sync_copy(spm.at[slab], hs_q.at[slab])
        idx_fetch(0, 0)
        plsc.subcore_barrier()

        for t in range(NSLICE):
            p = t % 2
            idx_wait(p)
            if t + 1 < NSLICE:
                idx_fetch(t + 1, 1 - p)
            src_v, dst_v = srcs[p], dsts[p]

            for b in range(NBUF):
                pltpu.async_copy(hs_q.at[src_v.at[b]], rows[b], gsems[b])

            @pl.loop(0, SROWS, step=NBUF)
            def _(j):
                for b in range(NBUF):
                    pltpu.make_async_copy(hs_q.at[src_v.at[j + b]], rows[b],
                                          gsems[b]).wait()
                    pltpu.async_copy(rows[b], spm.at[dst_v.at[j + b]],
                                     ssems[b], add=True)

                    @pl.when(j + b + NBUF < SROWS)
                    def _():
                        pltpu.make_async_copy(rows[b], spm.at[dst_v.at[0]],
                                              ssems[b]).wait()
                        pltpu.async_copy(hs_q.at[src_v.at[j + b + NBUF]],
                                         rows[b], gsems[b])

            for b in range(NBUF):
                pltpu.make_async_copy(rows[b], spm.at[dst_v.at[0]],
                                      ssems[b]).wait()

        plsc.subcore_barrier()
        pltpu.sync_copy(spm.at[slab], agg.at[slab, col])

    @pl.when(c == 0)
    def _():
        for q in range(NCH // 2):
            run_chunk(q)

    @pl.when(c == 1)
    def _():
        for q in range(NCH // 2, NCH):
            run_chunk(q)


def _sc_agg(hs, src_lin, dst_lin):
    st = jax.ShapeDtypeStruct((N_NODES, CW), _f32)
    fn = pl.kernel(
        _agg_body,
        out_type=(jax.ShapeDtypeStruct((N_NODES, FEAT), _f32),) + (st,) * NCH,
        mesh=plsc.VectorSubcoreMesh(core_axis_name="c", subcore_axis_name="s"),
        scratch_types=(
            [pltpu.VMEM((SROWS, IDXW), jnp.int32)] * 4
            + [pltpu.VMEM((IDXW, CW), _f32)] * NBUF
            + [pltpu.SemaphoreType.DMA] * (2 * NBUF + 2)
            + [pltpu.VMEM_SHARED((N_NODES, CW), _f32)]
        ),
        compiler_params=_SC_PARAMS,
    )
    return fn(hs, src_lin, dst_lin)[0]


def _init_kernel(zc_ref, wi_ref, bi_ref, wt_ref, g_ref):
    h = jnp.dot(zc_ref[...], wi_ref[...], preferred_element_type=_f32)
    h = jax.nn.relu(h + bi_ref[...])
    g_ref[...] = jnp.dot(h, wt_ref[...], preferred_element_type=_f32)


def _tc_init(zc, w_init, b_init, wc1_top):
    return pl.pallas_call(
        _init_kernel,
        out_shape=jax.ShapeDtypeStruct((N_GRAPHS, FEAT), _f32),
    )(zc, w_init, b_init, wc1_top)


def _dis_from_deg(degp_ref):
    deg = degp_ref[0, :, 0] + degp_ref[1, :, 0] + 1.0
    return lax.rsqrt(deg)


_WIDE_SPEC = pl.BlockSpec((ROW_TILE, FEAT), lambda i: (i, 0))
_DEG_SPEC = pl.BlockSpec((2, ROW_TILE, DEG_W), lambda i: (0, i, 0))


def _count_kernel(batch_ref, c_ref):
    t = pl.program_id(0)

    @pl.when(t == 0)
    def _():
        c_ref[...] = jnp.zeros_like(c_ref)

    bt = batch_ref[0, 0, :]
    giota = lax.broadcasted_iota(jnp.int32, (ROW_TILE, N_GRAPHS), 1)
    oh = (bt[:, None] == giota).astype(_f32)
    c_ref[...] += jnp.sum(oh, axis=0, keepdims=True)


def _tc_counts(batch3):
    return pl.pallas_call(
        _count_kernel,
        grid=(N_TILES,),
        in_specs=[pl.BlockSpec((1, 1, ROW_TILE), lambda i: (i, 0, 0))],
        out_specs=pl.BlockSpec((1, N_GRAPHS), lambda i: (0, 0)),
        out_shape=jax.ShapeDtypeStruct((1, N_GRAPHS), _f32),
    )(batch3)


def _build_kernel(batch_ref, degp_ref, g_ref, first_ref, woh_ref, o_ref):
    t = pl.program_id(0)
    bt = batch_ref[0, 0, :]
    giota = lax.broadcasted_iota(jnp.int32, (ROW_TILE, N_GRAPHS), 1)
    oh = (bt[:, None] == giota).astype(_f32)
    gexp = jnp.dot(oh, g_ref[...], preferred_element_type=_f32)
    fb = jnp.sum(oh * first_ref[0, :][None, :], axis=1)
    riota = lax.broadcasted_iota(jnp.int32, (ROW_TILE, 1), 0)[:, 0]
    order = t * ROW_TILE + riota - fb.astype(jnp.int32)
    oiota = lax.broadcasted_iota(jnp.int32, (ROW_TILE, FEAT), 1)
    oh2 = (order[:, None] == oiota).astype(_f32)
    wexp = jnp.dot(oh2, woh_ref[...], preferred_element_type=_f32)
    dis = _dis_from_deg(degp_ref)
    o_ref[...] = (gexp + wexp) * dis[:, None]


def _tc_build(batch3, degp, g, first_f, woh_pad):
    return pl.pallas_call(
        _build_kernel,
        grid=(N_TILES,),
        in_specs=[
            pl.BlockSpec((1, 1, ROW_TILE), lambda i: (i, 0, 0)),
            _DEG_SPEC,
            pl.BlockSpec((N_GRAPHS, FEAT), lambda i: (0, 0)),
            pl.BlockSpec((1, N_GRAPHS), lambda i: (0, 0)),
            pl.BlockSpec((FEAT, FEAT), lambda i: (0, 0)),
        ],
        out_specs=_WIDE_SPEC,
        out_shape=jax.ShapeDtypeStruct((N_NODES, FEAT), _f32),
    )(batch3, degp, g, first_f, woh_pad)


def _layer_kernel(a_ref, degp_ref, b_ref, w_ref, o_ref):
    dis = _dis_from_deg(degp_ref)
    x = jax.nn.relu(a_ref[...] * dis[:, None] + b_ref[...])
    o_ref[...] = (jnp.dot(x, w_ref[...], preferred_element_type=_f32)
                  * dis[:, None])


def _tc_layer(agg, degp, b_prev, w_next):
    return pl.pallas_call(
        _layer_kernel,
        grid=(N_TILES,),
        in_specs=[
            _WIDE_SPEC,
            _DEG_SPEC,
            pl.BlockSpec((FEAT,), lambda i: (0,)),
            pl.BlockSpec((FEAT, FEAT), lambda i: (0, 0)),
        ],
        out_specs=_WIDE_SPEC,
        out_shape=jax.ShapeDtypeStruct((N_NODES, FEAT), _f32),
    )(agg, degp, b_prev, w_next)


def _head_kernel(a_ref, degp_ref, b3_ref, wp_ref, bp_ref, wfp_ref, bfp_ref,
                 ws_ref, bs_ref, wfs_ref, bfs_ref, pos_ref, size_ref):
    dis = _dis_from_deg(degp_ref)
    h = jax.nn.relu(a_ref[...] * dis[:, None] + b3_ref[...])
    ph = jax.nn.relu(jnp.dot(h, wp_ref[...], preferred_element_type=_f32)
                     + bp_ref[...])
    pos_ref[...] = (jnp.dot(ph, wfp_ref[...], preferred_element_type=_f32)
                    + bfp_ref[...])
    sh = jax.nn.relu(jnp.dot(h, ws_ref[...], preferred_element_type=_f32)
                     + bs_ref[...])
    size_ref[...] = (jnp.dot(sh, wfs_ref[...], preferred_element_type=_f32)
                     + bfs_ref[...])


def _tc_head(agg, degp, b_c3, w_pos, b_pos, w_fcp, b_fcp, w_size, b_size,
             w_fcs, b_fcs):
    full = lambda *shape: pl.BlockSpec(shape, lambda i: (0,) * len(shape))
    return pl.pallas_call(
        _head_kernel,
        grid=(N_TILES,),
        in_specs=[
            _WIDE_SPEC,
            _DEG_SPEC,
            full(FEAT), full(FEAT, FEAT), full(FEAT), full(FEAT, 2), full(2),
            full(FEAT, FEAT), full(FEAT), full(FEAT, 2), full(2),
        ],
        out_specs=[
            pl.BlockSpec((ROW_TILE, 2), lambda i: (i, 0)),
            pl.BlockSpec((ROW_TILE, 2), lambda i: (i, 0)),
        ],
        out_shape=[
            jax.ShapeDtypeStruct((N_NODES, 2), _f32),
            jax.ShapeDtypeStruct((N_NODES, 2), _f32),
        ],
    )(agg, degp, b_c3, w_pos, b_pos, w_fcp, b_fcp, w_size, b_size,
      w_fcs, b_fcs)


def kernel(z, condition, edge_index, batch, W_init, b_init, W_c1, b_c1, W_c2,
           b_c2, W_c3, b_c3, W_pos, b_pos, W_fcp, b_fcp, W_size, b_size,
           W_fcs, b_fcs):
    zc = jnp.concatenate([z, condition], axis=1)
    wc1_top = W_c1[:FEAT]
    woh_pad = jnp.zeros((FEAT, FEAT), _f32).at[:MAXN].set(W_c1[FEAT:])
    batch3 = batch.reshape(N_TILES, 1, ROW_TILE)
    counts = _tc_counts(batch3)
    first_f = (jnp.cumsum(counts[0]) - counts[0]).reshape(1, N_GRAPHS)
    src_lin = edge_index[0].reshape(N_EDGES // IDXW, IDXW)
    dst_lin = edge_index[1].reshape(N_EDGES // IDXW, IDXW)
    dst_deg = edge_index[1].reshape(N_EDGES // DIDXW, DIDXW)
    ones16 = jnp.zeros((DIDXW, DEG_W), _f32).at[:, 0].set(1.0)
    zeros16 = jnp.zeros((N_NODES, DEG_W), _f32)

    g = _tc_init(zc, W_init, b_init, wc1_top)
    degp = _sc_deg(dst_deg, ones16, zeros16)
    hs1 = _tc_build(batch3, degp, g, first_f, woh_pad)
    agg1 = _sc_agg(hs1, src_lin, dst_lin)
    hs2 = _tc_layer(agg1, degp, b_c1, W_c2)
    agg2 = _sc_agg(hs2, src_lin, dst_lin)
    hs3 = _tc_layer(agg2, degp, b_c2, W_c3)
    agg3 = _sc_agg(hs3, src_lin, dst_lin)
    pos, size = _tc_head(agg3, degp, b_c3, W_pos, b_pos, W_fcp, b_fcp,
                         W_size, b_size, W_fcs, b_fcs)
    return (pos, size)


# speedup vs baseline: 1.0918x; 1.0918x over previous
import functools

import jax
import jax.numpy as jnp
from jax import lax
from jax.experimental import pallas as pl
from jax.experimental.pallas import tpu as pltpu
from jax.experimental.pallas import tpu_sc as plsc

N_NODES = 61440
N_GRAPHS = 1024
N_EDGES = 614400
FEAT = 128
MAXN = 120

NSUB = 16
LANE = 128
IDXW = 128
EROWS = N_EDGES // (NSUB * IDXW)
NSLICE = 3
SROWS = EROWS // NSLICE
NBUF = 5
DIDXW = 128
DROWS = N_EDGES // (2 * NSUB * DIDXW)
NSLAB = N_NODES // NSUB
CW = 16
NCH = FEAT // CW
ROW_TILE = 1024
N_TILES = N_NODES // ROW_TILE
DEG_W = 16

_f32 = jnp.float32
_SC_PARAMS = pltpu.CompilerParams(use_tc_tiling_on_sc=False)


def _deg_body(dst_hbm, ones_hbm, zeros_hbm, deg_out, dst_v, ones_v, spm):
    c = lax.axis_index("c")
    s = lax.axis_index("s")
    w = c * NSUB + s
    pltpu.sync_copy(ones_hbm, ones_v)
    pltpu.sync_copy(dst_hbm.at[pl.ds(w * DROWS, DROWS)], dst_v)
    pltpu.sync_copy(zeros_hbm.at[pl.ds(s * NSLAB, NSLAB)],
                    spm.at[pl.ds(s * NSLAB, NSLAB)])
    plsc.subcore_barrier()

    @pl.loop(0, DROWS)
    def _(j):
        pltpu.sync_copy(ones_v, spm.at[dst_v.at[j]], add=True)

    plsc.subcore_barrier()
    pltpu.sync_copy(spm.at[pl.ds(s * NSLAB, NSLAB)],
                    deg_out.at[c, pl.ds(s * NSLAB, NSLAB)])


def _sc_deg(dst_lin, ones16, zeros16):
    fn = pl.kernel(
        _deg_body,
        out_type=jax.ShapeDtypeStruct((2, N_NODES, DEG_W), _f32),
        mesh=plsc.VectorSubcoreMesh(core_axis_name="c", subcore_axis_name="s"),
        scratch_types=[
            pltpu.VMEM((DROWS, DIDXW), jnp.int32),
            pltpu.VMEM((DIDXW, DEG_W), _f32),
            pltpu.VMEM_SHARED((N_NODES, DEG_W), _f32),
        ],
        compiler_params=_SC_PARAMS,
    )
    return fn(dst_lin, ones16, zeros16)


def _agg_body(hs, src_hbm, dst_hbm, agg, st0, st1, st2, st3, st4, st5, st6,
              st7,
              src_a, src_b, dst_a, dst_b, r0, r1, r2, r3, r4,
              sg0, sg1, sg2, sg3, sg4, ss0, ss1, ss2, ss3, ss4, si_a, si_b,
              spm):
    c = lax.axis_index("c")
    s = lax.axis_index("s")
    slab = pl.ds(s * NSLAB, NSLAB)
    stages = (st0, st1, st2, st3, st4, st5, st6, st7)
    srcs = (src_a, src_b)
    dsts = (dst_a, dst_b)
    isems = (si_a, si_b)
    rows = (r0, r1, r2, r3, r4)
    gsems = (sg0, sg1, sg2, sg3, sg4)
    ssems = (ss0, ss1, ss2, ss3, ss4)

    def idx_fetch(t, p):
        base = pl.ds(s * EROWS + t * SROWS, SROWS)
        pltpu.async_copy(src_hbm.at[base], srcs[p], isems[p])
        pltpu.async_copy(dst_hbm.at[base], dsts[p], isems[p])

    def idx_wait(p):
        pltpu.make_async_copy(src_hbm.at[pl.ds(0, SROWS)], srcs[p],
                              isems[p]).wait()
        pltpu.make_async_copy(dst_hbm.at[pl.ds(0, SROWS)], dsts[p],
                              isems[p]).wait()

    def run_chunk(q):
        col = pl.ds(q * CW, CW)
        hs_q = stages[q]
        pltpu.sync_copy(hs.at[slab, col], spm.at[slab])
        pltpu.sync_copy(spm.at[slab], hs_q.at[slab])
        idx_fetch(0, 0)
        plsc.subcore_barrier()

        for t in range(NSLICE):
            p = t % 2
            idx_wait(p)
            if t + 1 < NSLICE:
                idx_fetch(t + 1, 1 - p)
            src_v, dst_v = srcs[p], dsts[p]

            for b in range(NBUF):
                pltpu.async_copy(hs_q.at[src_v.at[b]], rows[b], gsems[b])

            @pl.loop(0, SROWS, step=NBUF)
            def _(j):
                for b in range(NBUF):
                    pltpu.make_async_copy(hs_q.at[src_v.at[j + b]], rows[b],
                                          gsems[b]).wait()
                    pltpu.async_copy(rows[b], spm.at[dst_v.at[j + b]],
                                     ssems[b], add=True)

                    @pl.when(j + b + NBUF < SROWS)
                    def _():
                        pltpu.make_async_copy(rows[b], spm.at[dst_v.at[0]],
                                              ssems[b]).wait()
                        pltpu.async_copy(hs_q.at[src_v.at[j + b + NBUF]],
                                         rows[b], gsems[b])

            for b in range(NBUF):
                pltpu.make_async_copy(rows[b], spm.at[dst_v.at[0]],
                                      ssems[b]).wait()

        plsc.subcore_barrier()
        pltpu.sync_copy(spm.at[slab], agg.at[slab, col])

    @pl.when(c == 0)
    def _():
        for q in range(NCH // 2):
            run_chunk(q)

    @pl.when(c == 1)
    def _():
        for q in range(NCH // 2, NCH):
            run_chunk(q)


def _sc_agg(hs, src_lin, dst_lin):
    st = jax.ShapeDtypeStruct((N_NODES, CW), _f32)
    fn = pl.kernel(
        _agg_body,
        out_type=(jax.ShapeDtypeStruct((N_NODES, FEAT), _f32),) + (st,) * NCH,
        mesh=plsc.VectorSubcoreMesh(core_axis_name="c", subcore_axis_name="s"),
        scratch_types=(
            [pltpu.VMEM((SROWS, IDXW), jnp.int32)] * 4
            + [pltpu.VMEM((IDXW, CW), _f32)] * NBUF
            + [pltpu.SemaphoreType.DMA] * (2 * NBUF + 2)
            + [pltpu.VMEM_SHARED((N_NODES, CW), _f32)]
        ),
        compiler_params=_SC_PARAMS,
    )
    return fn(hs, src_lin, dst_lin)[0]


def _init_kernel(zc_ref, wi_ref, bi_ref, wt_ref, g_ref):
    h = jnp.dot(zc_ref[...], wi_ref[...], preferred_element_type=_f32)
    h = jax.nn.relu(h + bi_ref[...])
    g_ref[...] = jnp.dot(h, wt_ref[...], preferred_element_type=_f32)


def _tc_init(zc, w_init, b_init, wc1_top):
    return pl.pallas_call(
        _init_kernel,
        out_shape=jax.ShapeDtypeStruct((N_GRAPHS, FEAT), _f32),
    )(zc, w_init, b_init, wc1_top)


def _dis_from_deg(degp_ref):
    deg = degp_ref[0, :, 0] + degp_ref[1, :, 0] + 1.0
    return lax.rsqrt(deg)


_WIDE_SPEC = pl.BlockSpec((ROW_TILE, FEAT), lambda i: (i, 0))
_DEG_SPEC = pl.BlockSpec((2, ROW_TILE, DEG_W), lambda i: (0, i, 0))


def _count_kernel(batch_ref, c_ref):
    t = pl.program_id(0)

    @pl.when(t == 0)
    def _():
        c_ref[...] = jnp.zeros_like(c_ref)

    bt = batch_ref[0, 0, :]
    giota = lax.broadcasted_iota(jnp.int32, (ROW_TILE, N_GRAPHS), 1)
    oh = (bt[:, None] == giota).astype(_f32)
    c_ref[...] += jnp.sum(oh, axis=0, keepdims=True)


def _tc_counts(batch3):
    return pl.pallas_call(
        _count_kernel,
        grid=(N_TILES,),
        in_specs=[pl.BlockSpec((1, 1, ROW_TILE), lambda i: (i, 0, 0))],
        out_specs=pl.BlockSpec((1, N_GRAPHS), lambda i: (0, 0)),
        out_shape=jax.ShapeDtypeStruct((1, N_GRAPHS), _f32),
    )(batch3)


def _build_kernel(batch_ref, degp_ref, g_ref, first_ref, woh_ref, o_ref):
    t = pl.program_id(0)
    bt = batch_ref[0, 0, :]
    giota = lax.broadcasted_iota(jnp.int32, (ROW_TILE, N_GRAPHS), 1)
    oh = (bt[:, None] == giota).astype(_f32)
    gexp = jnp.dot(oh, g_ref[...], preferred_element_type=_f32)
    fb = jnp.sum(oh * first_ref[0, :][None, :], axis=1)
    riota = lax.broadcasted_iota(jnp.int32, (ROW_TILE, 1), 0)[:, 0]
    order = t * ROW_TILE + riota - fb.astype(jnp.int32)
    oiota = lax.broadcasted_iota(jnp.int32, (ROW_TILE, FEAT), 1)
    oh2 = (order[:, None] == oiota).astype(_f32)
    wexp = jnp.dot(oh2, woh_ref[...], preferred_element_type=_f32)
    dis = _dis_from_deg(degp_ref)
    o_ref[...] = (gexp + wexp) * dis[:, None]


def _tc_build(batch3, degp, g, first_f, woh_pad):
    return pl.pallas_call(
        _build_kernel,
        grid=(N_TILES,),
        in_specs=[
            pl.BlockSpec((1, 1, ROW_TILE), lambda i: (i, 0, 0)),
            _DEG_SPEC,
            pl.BlockSpec((N_GRAPHS, FEAT), lambda i: (0, 0)),
            pl.BlockSpec((1, N_GRAPHS), lambda i: (0, 0)),
            pl.BlockSpec((FEAT, FEAT), lambda i: (0, 0)),
        ],
        out_specs=_WIDE_SPEC,
        out_shape=jax.ShapeDtypeStruct((N_NODES, FEAT), _f32),
    )(batch3, degp, g, first_f, woh_pad)


def _layer_kernel(a_ref, degp_ref, b_ref, w_ref, o_ref):
    dis = _dis_from_deg(degp_ref)
    x = jax.nn.relu(a_ref[...] * dis[:, None] + b_ref[...])
    o_ref[...] = (jnp.dot(x, w_ref[...], preferred_element_type=_f32)
                  * dis[:, None])


def _tc_layer(agg, degp, b_prev, w_next):
    return pl.pallas_call(
        _layer_kernel,
        grid=(N_TILES,),
        in_specs=[
            _WIDE_SPEC,
            _DEG_SPEC,
            pl.BlockSpec((FEAT,), lambda i: (0,)),
            pl.BlockSpec((FEAT, FEAT), lambda i: (0, 0)),
        ],
        out_specs=_WIDE_SPEC,
        out_shape=jax.ShapeDtypeStruct((N_NODES, FEAT), _f32),
    )(agg, degp, b_prev, w_next)


def _head_kernel(a_ref, degp_ref, b3_ref, wp_ref, bp_ref, wfp_ref, bfp_ref,
                 ws_ref, bs_ref, wfs_ref, bfs_ref, pos_ref, size_ref):
    dis = _dis_from_deg(degp_ref)
    h = jax.nn.relu(a_ref[...] * dis[:, None] + b3_ref[...])
    ph = jax.nn.relu(jnp.dot(h, wp_ref[...], preferred_element_type=_f32)
                     + bp_ref[...])
    pos_ref[...] = (jnp.dot(ph, wfp_ref[...], preferred_element_type=_f32)
                    + bfp_ref[...])
    sh = jax.nn.relu(jnp.dot(h, ws_ref[...], preferred_element_type=_f32)
                     + bs_ref[...])
    size_ref[...] = (jnp.dot(sh, wfs_ref[...], preferred_element_type=_f32)
                     + bfs_ref[...])


def _tc_head(agg, degp, b_c3, w_pos, b_pos, w_fcp, b_fcp, w_size, b_size,
             w_fcs, b_fcs):
    full = lambda *shape: pl.BlockSpec(shape, lambda i: (0,) * len(shape))
    return pl.pallas_call(
        _head_kernel,
        grid=(N_TILES,),
        in_specs=[
            _WIDE_SPEC,
            _DEG_SPEC,
            full(FEAT), full(FEAT, FEAT), full(FEAT), full(FEAT, 2), full(2),
            full(FEAT, FEAT), full(FEAT), full(FEAT, 2), full(2),
        ],
        out_specs=[
            pl.BlockSpec((ROW_TILE, 2), lambda i: (i, 0)),
            pl.BlockSpec((ROW_TILE, 2), lambda i: (i, 0)),
        ],
        out_shape=[
            jax.ShapeDtypeStruct((N_NODES, 2), _f32),
            jax.ShapeDtypeStruct((N_NODES, 2), _f32),
        ],
    )(agg, degp, b_c3, w_pos, b_pos, w_fcp, b_fcp, w_size, b_size,
      w_fcs, b_fcs)


def kernel(z, condition, edge_index, batch, W_init, b_init, W_c1, b_c1, W_c2,
           b_c2, W_c3, b_c3, W_pos, b_pos, W_fcp, b_fcp, W_size, b_size,
           W_fcs, b_fcs):
    zc = jnp.concatenate([z, condition], axis=1)
    wc1_top = W_c1[:FEAT]
    woh_pad = jnp.zeros((FEAT, FEAT), _f32).at[:MAXN].set(W_c1[FEAT:])
    batch3 = batch.reshape(N_TILES, 1, ROW_TILE)
    counts = _tc_counts(batch3)
    first_f = (jnp.cumsum(counts[0]) - counts[0]).reshape(1, N_GRAPHS)
    src_lin = edge_index[0].reshape(N_EDGES // IDXW, IDXW)
    dst_lin = edge_index[1].reshape(N_EDGES // IDXW, IDXW)
    dst_deg = edge_index[1].reshape(N_EDGES // DIDXW, DIDXW)
    ones16 = jnp.zeros((DIDXW, DEG_W), _f32).at[:, 0].set(1.0)
    zeros16 = jnp.zeros((N_NODES, DEG_W), _f32)

    g = _tc_init(zc, W_init, b_init, wc1_top)
    degp = _sc_deg(dst_deg, ones16, zeros16)
    hs1 = _tc_build(batch3, degp, g, first_f, woh_pad)
    agg1 = _sc_agg(hs1, src_lin, dst_lin)
    hs2 = _tc_layer(agg1, degp, b_c1, W_c2)
    agg2 = _sc_agg(hs2, src_lin, dst_lin)
    hs3 = _tc_layer(agg2, degp, b_c2, W_c3)
    agg3 = _sc_agg(hs3, src_lin, dst_lin)
    pos, size = _tc_head(agg3, degp, b_c3, W_pos, b_pos, W_fcp, b_fcp,
                         W_size, b_size, W_fcs, b_fcs)
    return (pos, size)
